# Initial kernel scaffold; baseline (speedup 1.0000x reference)
import functools

import jax
import jax.numpy as jnp
from jax import lax
from jax.experimental import pallas as pl
from jax.experimental.pallas import tpu as pltpu
from jax.experimental.pallas import tpu_sc as plsc

F32 = jnp.float32
EPS = 1e-5

NC = 2
NS = 16
NW = NC * NS
CH = 128

_HIGH = jax.lax.Precision.HIGHEST


def _dot(a, b):
    return jnp.dot(a, b, preferred_element_type=F32, precision=_HIGH)


def _silu(v):
    return v * jax.nn.sigmoid(v)


def _prep_body(h, w1a, w1b, be1, t1_o, t2_o):
    hb = h[...]
    t1_o[...] = _dot(hb, w1a[...])
    t2_o[...] = _dot(hb, w1b[...]) + be1[...]


def _prep(h, w1a, w1b, be1, bn):
    n, d = h.shape
    grid = n // bn
    return pl.pallas_call(
        _prep_body,
        grid=(grid,),
        in_specs=[
            pl.BlockSpec((bn, d), lambda i: (i, 0)),
            pl.BlockSpec((d, d), lambda i: (0, 0)),
            pl.BlockSpec((d, d), lambda i: (0, 0)),
            pl.BlockSpec((1, d), lambda i: (0, 0)),
        ],
        out_specs=[
            pl.BlockSpec((bn, d), lambda i: (i, 0)),
            pl.BlockSpec((bn, d), lambda i: (i, 0)),
        ],
        out_shape=[
            jax.ShapeDtypeStruct((n, d), F32),
            jax.ShapeDtypeStruct((n, d), F32),
        ],
    )(h, w1a, w1b, be1)


def _sc_gather(t1, t2, xt, srcg, dstg, e_pad):
    n, d = t1.shape
    xw = xt.shape[1]
    epw = e_pad // NW
    nch = epw // CH
    mesh = plsc.VectorSubcoreMesh(core_axis_name="c", subcore_axis_name="s")

    @functools.partial(
        pl.kernel,
        mesh=mesh,
        out_type=[
            jax.ShapeDtypeStruct((e_pad, d), F32),
            jax.ShapeDtypeStruct((e_pad, d), F32),
            jax.ShapeDtypeStruct((e_pad, xw), F32),
            jax.ShapeDtypeStruct((e_pad, xw), F32),
        ],
        scratch_types=[
            pltpu.VMEM((CH,), jnp.int32),
            pltpu.VMEM((CH,), jnp.int32),
            pltpu.VMEM((CH, d), F32),
            pltpu.VMEM((CH, d), F32),
            pltpu.VMEM((CH, xw), F32),
            pltpu.VMEM((CH, xw), F32),
            pltpu.SemaphoreType.DMA,
            pltpu.SemaphoreType.DMA,
            pltpu.SemaphoreType.DMA,
            pltpu.SemaphoreType.DMA,
        ],
    )
    def k(t1h, t2h, xth, sh, dh, g1o, g2o, xso, xdo,
          sv, dv, g1v, g2v, xsv, xdv, m1, m2, m3, m4):
        cid = lax.axis_index("c")
        sid = lax.axis_index("s")
        base = (cid * NS + sid) * epw

        @pl.loop(0, nch)
        def _(j):
            off = base + j * CH
            pltpu.sync_copy(sh.at[pl.ds(off, CH)], sv)
            pltpu.sync_copy(dh.at[pl.ds(off, CH)], dv)
            c1 = pltpu.async_copy(t1h.at[sv], g1v, m1)
            c2 = pltpu.async_copy(t2h.at[dv], g2v, m2)
            c3 = pltpu.async_copy(xth.at[sv], xsv, m3)
            c4 = pltpu.async_copy(xth.at[dv], xdv, m4)
            c1.wait()
            c2.wait()
            c3.wait()
            c4.wait()
            pltpu.sync_copy(g1v, g1o.at[pl.ds(off, CH)])
            pltpu.sync_copy(g2v, g2o.at[pl.ds(off, CH)])
            pltpu.sync_copy(xsv, xso.at[pl.ds(off, CH)])
            pltpu.sync_copy(xdv, xdo.at[pl.ds(off, CH)])

    return k(t1, t2, xt, srcg, dstg)


def _edge_body(g1, g2, xs, xd, ea, wdr, w1c, we2, be2, wc1, bc1, wc2r, bc2r,
               msg_o, swd_o):
    rel = xs[...] - xd[...]
    ds = jnp.sum(rel * rel, axis=1, keepdims=True)
    pre = g1[...] + g2[...] + ds * wdr[...] + _dot(ea[...], w1c[...])
    m = _silu(pre)
    msg = _silu(_dot(m, we2[...]) + be2[...])
    t = _silu(_dot(msg, wc1[...]) + bc1[...])
    cw = jnp.sum(t * wc2r[...], axis=1, keepdims=True) + bc2r[0:1, 0:1]
    msg_o[...] = msg
    swd_o[...] = rel * cw


def _edge(g1, g2, xs, xd, ea, wdr, w1c, we2, be2, wc1, bc1, wc2r, bc2r,
          e, be_blk):
    e_pad, d = g1.shape
    xw = xs.shape[1]
    ed = ea.shape[1]
    grid = e // be_blk

    def eb(i):
        return (i, 0)

    def cb(i):
        return (0, 0)

    return pl.pallas_call(
        _edge_body,
        grid=(grid,),
        in_specs=[
            pl.BlockSpec((be_blk, d), eb),
            pl.BlockSpec((be_blk, d), eb),
            pl.BlockSpec((be_blk, xw), eb),
            pl.BlockSpec((be_blk, xw), eb),
            pl.BlockSpec((be_blk, ed), eb),
            pl.BlockSpec((1, d), cb),
            pl.BlockSpec((ed, d), cb),
            pl.BlockSpec((d, d), cb),
            pl.BlockSpec((1, d), cb),
            pl.BlockSpec((d, d), cb),
            pl.BlockSpec((1, d), cb),
            pl.BlockSpec((1, d), cb),
            pl.BlockSpec((1, d), cb),
        ],
        out_specs=[
            pl.BlockSpec((be_blk, d), eb),
            pl.BlockSpec((be_blk, xw), eb),
        ],
        out_shape=[
            jax.ShapeDtypeStruct((e_pad, d), F32),
            jax.ShapeDtypeStruct((e_pad, xw), F32),
        ],
    )(g1, g2, xs, xd, ea, wdr, w1c, we2, be2, wc1, bc1, wc2r, bc2r)


def _sc_scatter(msg, swd, dsts, za, zx, na):
    e_pad, d = msg.shape
    xw = swd.shape[1]
    epw = e_pad // NW
    nch = epw // CH
    rpt = na // NS
    mesh = plsc.VectorSubcoreMesh(core_axis_name="c", subcore_axis_name="s")

    @functools.partial(
        pl.kernel,
        mesh=mesh,
        out_type=[
            jax.ShapeDtypeStruct((NC * na, d), F32),
            jax.ShapeDtypeStruct((NC * na, xw), F32),
        ],
        scratch_types=[
            pltpu.VMEM_SHARED((na, d), F32),
            pltpu.VMEM_SHARED((na, xw), F32),
            pltpu.VMEM((CH, d), F32),
            pltpu.VMEM((CH, xw), F32),
            pltpu.VMEM((CH,), jnp.int32),
        ],
    )
    def k(msgh, swdh, dsth, zah, zxh, aggo, xco, aggsh, xcsh, mv, wv, dv):
        cid = lax.axis_index("c")
        sid = lax.axis_index("s")
        wid = cid * NS + sid
        r0 = sid * rpt
        pltpu.sync_copy(zah.at[pl.ds(r0, rpt)], aggsh.at[pl.ds(r0, rpt)])
        pltpu.sync_copy(zxh.at[pl.ds(r0, rpt)], xcsh.at[pl.ds(r0, rpt)])
        plsc.subcore_barrier()

        @pl.loop(0, nch)
        def _(j):
            off = wid * epw + j * CH
            pltpu.sync_copy(dsth.at[pl.ds(off, CH)], dv)
            pltpu.sync_copy(msgh.at[pl.ds(off, CH)], mv)
            pltpu.sync_copy(swdh.at[pl.ds(off, CH)], wv)
            pltpu.sync_copy(mv, aggsh.at[dv], add=True)
            pltpu.sync_copy(wv, xcsh.at[dv], add=True)

        plsc.subcore_barrier()
        o0 = cid * na + r0
        pltpu.sync_copy(aggsh.at[pl.ds(r0, rpt)], aggo.at[pl.ds(o0, rpt)])
        pltpu.sync_copy(xcsh.at[pl.ds(r0, rpt)], xco.at[pl.ds(o0, rpt)])

    return k(msg, swd, dsts, za, zx)


def _node_body(h, a0, a1, xp, xc0, xc1, wn1a, wn1b, bn1, wn2, bn2, lng, lnb,
               ho_o, xo_o):
    hb = h[...]
    agg = a0[...] + a1[...]
    u = _silu(_dot(hb, wn1a[...]) + _dot(agg, wn1b[...]) + bn1[...])
    ho = hb + _dot(u, wn2[...]) + bn2[...]
    mu = jnp.mean(ho, axis=1, keepdims=True)
    dev = ho - mu
    var = jnp.mean(dev * dev, axis=1, keepdims=True)
    ho_o[...] = dev * jax.lax.rsqrt(var + EPS) * lng[...] + lnb[...]
    xo_o[...] = xp[...] + xc0[...] + xc1[...]


def _node(h, a0, a1, xp, xc0, xc1, wn1a, wn1b, bn1, wn2, bn2, lng, lnb, bn):
    n, d = h.shape
    xw = xp.shape[1]
    grid = n // bn

    def nb(i):
        return (i, 0)

    def cb(i):
        return (0, 0)

    return pl.pallas_call(
        _node_body,
        grid=(grid,),
        in_specs=[
            pl.BlockSpec((bn, d), nb),
            pl.BlockSpec((bn, d), nb),
            pl.BlockSpec((bn, d), nb),
            pl.BlockSpec((bn, xw), nb),
            pl.BlockSpec((bn, xw), nb),
            pl.BlockSpec((bn, xw), nb),
            pl.BlockSpec((d, d), cb),
            pl.BlockSpec((d, d), cb),
            pl.BlockSpec((1, d), cb),
            pl.BlockSpec((d, d), cb),
            pl.BlockSpec((1, d), cb),
            pl.BlockSpec((1, d), cb),
            pl.BlockSpec((1, d), cb),
        ],
        out_specs=[
            pl.BlockSpec((bn, d), nb),
            pl.BlockSpec((bn, xw), nb),
        ],
        out_shape=[
            jax.ShapeDtypeStruct((n, d), F32),
            jax.ShapeDtypeStruct((n, xw), F32),
        ],
    )(h, a0, a1, xp, xc0, xc1, wn1a, wn1b, bn1, wn2, bn2, lng, lnb)


def kernel(h, x, edge_index, edge_attr, W_e1, b_e1, W_e2, b_e2, W_n1, b_n1,
           W_n2, b_n2, W_c1, b_c1, W_c2, b_c2, ln_g, ln_b):
    n, d = h.shape
    e = edge_index.shape[1]
    xw = 16

    e_pad = ((e + NW * CH - 1) // (NW * CH)) * (NW * CH)
    na = ((n + 1 + NS - 1) // NS) * NS

    src = edge_index[0]
    dst = edge_index[1]
    padg = jnp.zeros((e_pad - e,), jnp.int32)
    srcg = jnp.concatenate([src, padg])
    dstg = jnp.concatenate([dst, padg])
    dsts = jnp.concatenate([dst, jnp.full((e_pad - e,), n, jnp.int32)])

    w1a = W_e1[:d]
    w1b = W_e1[d:2 * d]
    wdr = W_e1[2 * d:2 * d + 1]
    w1c = W_e1[2 * d + 1:]
    be1 = b_e1.reshape(1, d)
    be2 = b_e2.reshape(1, d)
    bc1 = b_c1.reshape(1, d)
    wc2r = W_c2.reshape(1, d)
    bc2r = jnp.broadcast_to(b_c2.reshape(1, 1), (1, d))
    wn1a = W_n1[:d]
    wn1b = W_n1[d:]
    bn1 = b_n1.reshape(1, d)
    bn2 = b_n2.reshape(1, d)
    lng = ln_g.reshape(1, d)
    lnb = ln_b.reshape(1, d)

    xp = jnp.pad(x, ((0, 0), (0, xw - x.shape[1])))

    t1, t2 = _prep(h, w1a, w1b, be1, bn=1000)
    g1, g2, xs, xd = _sc_gather(t1, t2, xp, srcg, dstg, e_pad)
    msg, swd = _edge(g1, g2, xs, xd, edge_attr, wdr, w1c, W_e2, be2, W_c1,
                     bc1, wc2r, bc2r, e, be_blk=4000)
    za = jnp.zeros((na, d), F32)
    zx = jnp.zeros((na, xw), F32)
    aggp, xcp = _sc_scatter(msg, swd, dsts, za, zx, na)
    a0 = aggp[:n]
    a1 = aggp[na:na + n]
    xc0 = xcp[:n]
    xc1 = xcp[na:na + n]
    h_out, xo16 = _node(h, a0, a1, xp, xc0, xc1, wn1a, wn1b, bn1, W_n2, bn2,
                        lng, lnb, bn=1000)
    return (h_out, xo16[:, :x.shape[1]])


# speedup vs baseline: 1.8889x; 1.8889x over previous
import functools

import jax
import jax.numpy as jnp
from jax import lax
from jax.experimental import pallas as pl
from jax.experimental.pallas import tpu as pltpu
from jax.experimental.pallas import tpu_sc as plsc

F32 = jnp.float32
EPS = 1e-5

NC = 2
NS = 16
NW = NC * NS
CH = 128

_HIGH = jax.lax.Precision.HIGHEST


def _dot(a, b):
    return jnp.dot(a, b, preferred_element_type=F32, precision=_HIGH)


def _silu(v):
    return v * jax.nn.sigmoid(v)


def _prep_body(h, w1a, w1b, be1, t1_o, t2_o):
    hb = h[...]
    t1_o[...] = _dot(hb, w1a[...])
    t2_o[...] = _dot(hb, w1b[...]) + be1[...]


def _prep(h, w1a, w1b, be1, bn):
    n, d = h.shape
    grid = n // bn
    return pl.pallas_call(
        _prep_body,
        grid=(grid,),
        in_specs=[
            pl.BlockSpec((bn, d), lambda i: (i, 0)),
            pl.BlockSpec((d, d), lambda i: (0, 0)),
            pl.BlockSpec((d, d), lambda i: (0, 0)),
            pl.BlockSpec((1, d), lambda i: (0, 0)),
        ],
        out_specs=[
            pl.BlockSpec((bn, d), lambda i: (i, 0)),
            pl.BlockSpec((bn, d), lambda i: (i, 0)),
        ],
        out_shape=[
            jax.ShapeDtypeStruct((n, d), F32),
            jax.ShapeDtypeStruct((n, d), F32),
        ],
    )(h, w1a, w1b, be1)


def _sc_gather(t1, t2, xt, srcg, dstg, e_pad):
    n, d = t1.shape
    xw = xt.shape[1]
    epw = e_pad // NW
    nch = epw // CH
    mesh = plsc.VectorSubcoreMesh(core_axis_name="c", subcore_axis_name="s")

    @functools.partial(
        pl.kernel,
        mesh=mesh,
        out_type=[
            jax.ShapeDtypeStruct((e_pad, d), F32),
            jax.ShapeDtypeStruct((e_pad, d), F32),
            jax.ShapeDtypeStruct((e_pad, xw), F32),
            jax.ShapeDtypeStruct((e_pad, xw), F32),
        ],
        scratch_types=[
            pltpu.VMEM((CH,), jnp.int32),
            pltpu.VMEM((CH,), jnp.int32),
            pltpu.VMEM((CH, d), F32),
            pltpu.VMEM((CH, d), F32),
            pltpu.VMEM((CH, xw), F32),
            pltpu.VMEM((CH, xw), F32),
            pltpu.SemaphoreType.DMA,
            pltpu.SemaphoreType.DMA,
            pltpu.SemaphoreType.DMA,
            pltpu.SemaphoreType.DMA,
        ],
    )
    def k(t1h, t2h, xth, sh, dh, g1o, g2o, xso, xdo,
          sv, dv, g1v, g2v, xsv, xdv, m1, m2, m3, m4):
        cid = lax.axis_index("c")
        sid = lax.axis_index("s")
        base = (cid * NS + sid) * epw

        @pl.loop(0, nch)
        def _(j):
            off = base + j * CH
            pltpu.sync_copy(sh.at[pl.ds(off, CH)], sv)
            pltpu.sync_copy(dh.at[pl.ds(off, CH)], dv)
            c1 = pltpu.async_copy(t1h.at[sv], g1v, m1)
            c2 = pltpu.async_copy(t2h.at[dv], g2v, m2)
            c3 = pltpu.async_copy(xth.at[sv], xsv, m3)
            c4 = pltpu.async_copy(xth.at[dv], xdv, m4)
            c1.wait()
            c2.wait()
            c3.wait()
            c4.wait()
            pltpu.sync_copy(g1v, g1o.at[pl.ds(off, CH)])
            pltpu.sync_copy(g2v, g2o.at[pl.ds(off, CH)])
            pltpu.sync_copy(xsv, xso.at[pl.ds(off, CH)])
            pltpu.sync_copy(xdv, xdo.at[pl.ds(off, CH)])

    return k(t1, t2, xt, srcg, dstg)


def _edge_body(g1, g2, xs, xd, ea, wdr, w1c, we2, be2, wc1, bc1, wc2r, bc2r,
               msg_o, swd_o):
    rel = xs[...] - xd[...]
    ds = jnp.sum(rel * rel, axis=1, keepdims=True)
    pre = g1[...] + g2[...] + ds * wdr[...] + _dot(ea[...], w1c[...])
    m = _silu(pre)
    msg = _silu(_dot(m, we2[...]) + be2[...])
    t = _silu(_dot(msg, wc1[...]) + bc1[...])
    cw = jnp.sum(t * wc2r[...], axis=1, keepdims=True) + bc2r[0:1, 0:1]
    msg_o[...] = msg
    swd_o[...] = rel * cw


def _edge(g1, g2, xs, xd, ea, wdr, w1c, we2, be2, wc1, bc1, wc2r, bc2r,
          e, be_blk):
    e_pad, d = g1.shape
    ed = ea.shape[1]
    grid = e // be_blk

    def eb(i):
        return (i, 0)

    def cb(i):
        return (0, 0)

    return pl.pallas_call(
        _edge_body,
        grid=(grid,),
        in_specs=[
            pl.BlockSpec((be_blk, d), eb),
            pl.BlockSpec((be_blk, d), eb),
            pl.BlockSpec((be_blk, d), eb),
            pl.BlockSpec((be_blk, d), eb),
            pl.BlockSpec((be_blk, ed), eb),
            pl.BlockSpec((1, d), cb),
            pl.BlockSpec((ed, d), cb),
            pl.BlockSpec((d, d), cb),
            pl.BlockSpec((1, d), cb),
            pl.BlockSpec((d, d), cb),
            pl.BlockSpec((1, d), cb),
            pl.BlockSpec((1, d), cb),
            pl.BlockSpec((1, d), cb),
        ],
        out_specs=[
            pl.BlockSpec((be_blk, d), eb),
            pl.BlockSpec((be_blk, d), eb),
        ],
        out_shape=[
            jax.ShapeDtypeStruct((e_pad, d), F32),
            jax.ShapeDtypeStruct((e_pad, d), F32),
        ],
    )(g1, g2, xs, xd, ea, wdr, w1c, we2, be2, wc1, bc1, wc2r, bc2r)


def _sc_scatter(msg, swd, dsts, za, na):
    e_pad, d = msg.shape
    epc = e_pad // NS
    nch = epc // CH
    rpt = na // NS
    mesh = plsc.VectorSubcoreMesh(core_axis_name="c", subcore_axis_name="s")

    @functools.partial(
        pl.kernel,
        mesh=mesh,
        out_type=[
            jax.ShapeDtypeStruct((na, d), F32),
            jax.ShapeDtypeStruct((na, d), F32),
        ],
        scratch_types=[
            pltpu.VMEM_SHARED((na, d), F32),
            pltpu.VMEM((CH, d), F32),
            pltpu.VMEM((CH,), jnp.int32),
        ],
    )
    def k(msgh, swdh, dsth, zah, aggo, xco, accsh, mv, dv):
        cid = lax.axis_index("c")
        sid = lax.axis_index("s")
        r0 = sid * rpt
        pltpu.sync_copy(zah.at[pl.ds(r0, rpt)], accsh.at[pl.ds(r0, rpt)])
        plsc.subcore_barrier()

        @pl.when(cid == 0)
        def _():
            @pl.loop(0, nch)
            def _(j):
                off = sid * epc + j * CH
                pltpu.sync_copy(dsth.at[pl.ds(off, CH)], dv)
                pltpu.sync_copy(msgh.at[pl.ds(off, CH)], mv)
                pltpu.sync_copy(mv, accsh.at[dv], add=True)

        @pl.when(cid == 1)
        def _():
            @pl.loop(0, nch)
            def _(j):
                off = sid * epc + j * CH
                pltpu.sync_copy(dsth.at[pl.ds(off, CH)], dv)
                pltpu.sync_copy(swdh.at[pl.ds(off, CH)], mv)
                pltpu.sync_copy(mv, accsh.at[dv], add=True)

        plsc.subcore_barrier()

        @pl.when(cid == 0)
        def _():
            pltpu.sync_copy(accsh.at[pl.ds(r0, rpt)], aggo.at[pl.ds(r0, rpt)])

        @pl.when(cid == 1)
        def _():
            pltpu.sync_copy(accsh.at[pl.ds(r0, rpt)], xco.at[pl.ds(r0, rpt)])

    return k(msg, swd, dsts, za)


def _node_body(h, agg, xp, xc, wn1a, wn1b, bn1, wn2, bn2, lng, lnb,
               ho_o, xo_o):
    hb = h[...]
    u = _silu(_dot(hb, wn1a[...]) + _dot(agg[...], wn1b[...]) + bn1[...])
    ho = hb + _dot(u, wn2[...]) + bn2[...]
    mu = jnp.mean(ho, axis=1, keepdims=True)
    dev = ho - mu
    var = jnp.mean(dev * dev, axis=1, keepdims=True)
    ho_o[...] = dev * jax.lax.rsqrt(var + EPS) * lng[...] + lnb[...]
    xo_o[...] = xp[...] + xc[...][:, :16]


def _node(h, agg, xp, xc, wn1a, wn1b, bn1, wn2, bn2, lng, lnb, bn):
    n, d = h.shape
    xw = xp.shape[1]
    grid = n // bn

    def nb(i):
        return (i, 0)

    def cb(i):
        return (0, 0)

    return pl.pallas_call(
        _node_body,
        grid=(grid,),
        in_specs=[
            pl.BlockSpec((bn, d), nb),
            pl.BlockSpec((bn, d), nb),
            pl.BlockSpec((bn, xw), nb),
            pl.BlockSpec((bn, d), nb),
            pl.BlockSpec((d, d), cb),
            pl.BlockSpec((d, d), cb),
            pl.BlockSpec((1, d), cb),
            pl.BlockSpec((d, d), cb),
            pl.BlockSpec((1, d), cb),
            pl.BlockSpec((1, d), cb),
            pl.BlockSpec((1, d), cb),
        ],
        out_specs=[
            pl.BlockSpec((bn, d), nb),
            pl.BlockSpec((bn, xw), nb),
        ],
        out_shape=[
            jax.ShapeDtypeStruct((n, d), F32),
            jax.ShapeDtypeStruct((n, xw), F32),
        ],
    )(h, agg, xp, xc, wn1a, wn1b, bn1, wn2, bn2, lng, lnb)


def kernel(h, x, edge_index, edge_attr, W_e1, b_e1, W_e2, b_e2, W_n1, b_n1,
           W_n2, b_n2, W_c1, b_c1, W_c2, b_c2, ln_g, ln_b):
    n, d = h.shape
    e = edge_index.shape[1]
    xw = 16

    e_pad = ((e + NW * CH - 1) // (NW * CH)) * (NW * CH)
    na = ((n + 1 + NS * 8 - 1) // (NS * 8)) * (NS * 8)

    src = edge_index[0]
    dst = edge_index[1]
    padg = jnp.zeros((e_pad - e,), jnp.int32)
    srcg = jnp.concatenate([src, padg])
    dstg = jnp.concatenate([dst, padg])
    dsts = jnp.concatenate([dst, jnp.full((e_pad - e,), n, jnp.int32)])

    w1a = W_e1[:d]
    w1b = W_e1[d:2 * d]
    wdr = W_e1[2 * d:2 * d + 1]
    w1c = W_e1[2 * d + 1:]
    be1 = b_e1.reshape(1, d)
    be2 = b_e2.reshape(1, d)
    bc1 = b_c1.reshape(1, d)
    wc2r = W_c2.reshape(1, d)
    bc2r = jnp.broadcast_to(b_c2.reshape(1, 1), (1, d))
    wn1a = W_n1[:d]
    wn1b = W_n1[d:]
    bn1 = b_n1.reshape(1, d)
    bn2 = b_n2.reshape(1, d)
    lng = ln_g.reshape(1, d)
    lnb = ln_b.reshape(1, d)

    xp = jnp.pad(x, ((0, 0), (0, xw - x.shape[1])))
    xt = jnp.pad(x, ((0, 0), (0, d - x.shape[1])))

    t1, t2 = _prep(h, w1a, w1b, be1, bn=1000)
    g1, g2, xs, xd = _sc_gather(t1, t2, xt, srcg, dstg, e_pad)
    msg, swd = _edge(g1, g2, xs, xd, edge_attr, wdr, w1c, W_e2, be2, W_c1,
                     bc1, wc2r, bc2r, e, be_blk=4000)
    za = jnp.zeros((na, d), F32)
    agg, xc = _sc_scatter(msg, swd, dsts, za, na)
    h_out, xo16 = _node(h, agg[:n], xp, xc[:n], wn1a, wn1b, bn1, W_n2, bn2,
                        lng, lnb, bn=1000)
    return (h_out, xo16[:, :x.shape[1]])


# speedup vs baseline: 2.2191x; 1.1748x over previous
import dataclasses
import functools

import jax
import jax.numpy as jnp
from jax import lax
from jax.experimental import pallas as pl
from jax.experimental.pallas import tpu as pltpu
from jax.experimental.pallas import tpu_sc as plsc

F32 = jnp.float32
EPS = 1e-5

NC = 2
NS = 16
NW = NC * NS
CH = 128
SL = 16

_HIGH = jax.lax.Precision.HIGHEST


def _sc_params():
    cp = pltpu.CompilerParams()
    if "needs_layout_passes" in pltpu.CompilerParams.__dataclass_fields__:
        cp = dataclasses.replace(cp, needs_layout_passes=False)
    return cp


def _dot(a, b):
    return jnp.dot(a, b, preferred_element_type=F32, precision=_HIGH)


def _silu(v):
    return v * jax.nn.sigmoid(v)


def _prep_body(h, w1a, w1b, be1, t1_o, t2_o):
    hb = h[...]
    t1_o[...] = _dot(hb, w1a[...])
    t2_o[...] = _dot(hb, w1b[...]) + be1[...]


def _prep(h, w1a, w1b, be1, bn):
    n, d = h.shape
    grid = n // bn
    return pl.pallas_call(
        _prep_body,
        grid=(grid,),
        in_specs=[
            pl.BlockSpec((bn, d), lambda i: (i, 0)),
            pl.BlockSpec((d, d), lambda i: (0, 0)),
            pl.BlockSpec((d, d), lambda i: (0, 0)),
            pl.BlockSpec((1, d), lambda i: (0, 0)),
        ],
        out_specs=[
            pl.BlockSpec((bn, d), lambda i: (i, 0)),
            pl.BlockSpec((bn, d), lambda i: (i, 0)),
        ],
        out_shape=[
            jax.ShapeDtypeStruct((n, d), F32),
            jax.ShapeDtypeStruct((n, d), F32),
        ],
    )(h, w1a, w1b, be1)


def _sc_gather(t1, t2, xx, xy, xz, srcg, dstg, e_pad):
    n, d = t1.shape
    epw = e_pad // NW
    nch = epw // CH
    mesh = plsc.VectorSubcoreMesh(core_axis_name="c", subcore_axis_name="s")

    @functools.partial(
        pl.kernel,
        mesh=mesh,
        out_type=[
            jax.ShapeDtypeStruct((e_pad, d), F32),
            jax.ShapeDtypeStruct((e_pad, d), F32),
            jax.ShapeDtypeStruct((e_pad,), F32),
            jax.ShapeDtypeStruct((e_pad,), F32),
            jax.ShapeDtypeStruct((e_pad,), F32),
            jax.ShapeDtypeStruct((e_pad,), F32),
        ],
        scratch_types=[
            pltpu.VMEM((n,), F32),
            pltpu.VMEM((n,), F32),
            pltpu.VMEM((n,), F32),
            pltpu.VMEM((CH,), jnp.int32),
            pltpu.VMEM((CH,), jnp.int32),
            pltpu.VMEM((CH, d), F32),
            pltpu.VMEM((CH, d), F32),
            pltpu.VMEM((CH,), F32),
            pltpu.VMEM((CH,), F32),
            pltpu.VMEM((CH,), F32),
            pltpu.VMEM((CH,), F32),
            pltpu.SemaphoreType.DMA,
            pltpu.SemaphoreType.DMA,
        ],
        compiler_params=_sc_params(),
    )
    def k(t1h, t2h, xxh, xyh, xzh, sh, dh,
          g1o, g2o, rxo, ryo, rzo, dso,
          xxv, xyv, xzv, sv, dv, g1v, g2v, rxv, ryv, rzv, dsv, m1, m2):
        cid = lax.axis_index("c")
        sid = lax.axis_index("s")
        base = (cid * NS + sid) * epw
        pltpu.sync_copy(xxh, xxv)
        pltpu.sync_copy(xyh, xyv)
        pltpu.sync_copy(xzh, xzv)

        @pl.loop(0, nch)
        def _(j):
            off = base + j * CH
            pltpu.sync_copy(sh.at[pl.ds(off, CH)], sv)
            pltpu.sync_copy(dh.at[pl.ds(off, CH)], dv)
            c1 = pltpu.async_copy(t1h.at[sv], g1v, m1)
            c2 = pltpu.async_copy(t2h.at[dv], g2v, m2)
            for i in range(CH // SL):
                sl = pl.ds(i * SL, SL)
                si = sv[sl]
                di = dv[sl]
                rx = plsc.load_gather(xxv, [si]) - plsc.load_gather(xxv, [di])
                ry = plsc.load_gather(xyv, [si]) - plsc.load_gather(xyv, [di])
                rz = plsc.load_gather(xzv, [si]) - plsc.load_gather(xzv, [di])
                rxv[sl] = rx
                ryv[sl] = ry
                rzv[sl] = rz
                dsv[sl] = rx * rx + ry * ry + rz * rz
            c1.wait()
            c2.wait()
            pltpu.sync_copy(g1v, g1o.at[pl.ds(off, CH)])
            pltpu.sync_copy(g2v, g2o.at[pl.ds(off, CH)])
            pltpu.sync_copy(rxv, rxo.at[pl.ds(off, CH)])
            pltpu.sync_copy(ryv, ryo.at[pl.ds(off, CH)])
            pltpu.sync_copy(rzv, rzo.at[pl.ds(off, CH)])
            pltpu.sync_copy(dsv, dso.at[pl.ds(off, CH)])

    return k(t1, t2, xx, xy, xz, srcg, dstg)


def _edge_body(g1, g2, ea, ds1, wdr, w1c, we2, be2, wc1, bc1, wc2r, bc2r,
               msg_o, cw_o):
    dsb = ds1[...]
    dsc = jnp.reshape(dsb, (dsb.shape[0], 1))
    pre = g1[...] + g2[...] + dsc * wdr[...] + _dot(ea[...], w1c[...])
    m = _silu(pre)
    msg = _silu(_dot(m, we2[...]) + be2[...])
    t = _silu(_dot(msg, wc1[...]) + bc1[...])
    cw = jnp.sum(t * wc2r[...], axis=1) + bc2r[0, 0]
    msg_o[...] = msg
    cw_o[...] = cw


def _edge(g1, g2, ea, ds1, wdr, w1c, we2, be2, wc1, bc1, wc2r, bc2r,
          be_blk):
    e_pad, d = g1.shape
    ed = ea.shape[1]
    grid = e_pad // be_blk

    def eb(i):
        return (i, 0)

    def e1(i):
        return (i,)

    def cb(i):
        return (0, 0)

    return pl.pallas_call(
        _edge_body,
        grid=(grid,),
        in_specs=[
            pl.BlockSpec((be_blk, d), eb),
            pl.BlockSpec((be_blk, d), eb),
            pl.BlockSpec((be_blk, ed), eb),
            pl.BlockSpec((be_blk,), e1),
            pl.BlockSpec((1, d), cb),
            pl.BlockSpec((ed, d), cb),
            pl.BlockSpec((d, d), cb),
            pl.BlockSpec((1, d), cb),
            pl.BlockSpec((d, d), cb),
            pl.BlockSpec((1, d), cb),
            pl.BlockSpec((1, d), cb),
            pl.BlockSpec((1, d), cb),
        ],
        out_specs=[
            pl.BlockSpec((be_blk, d), eb),
            pl.BlockSpec((be_blk,), e1),
        ],
        out_shape=[
            jax.ShapeDtypeStruct((e_pad, d), F32),
            jax.ShapeDtypeStruct((e_pad,), F32),
        ],
    )(g1, g2, ea, ds1, wdr, w1c, we2, be2, wc1, bc1, wc2r, bc2r)


def _sc_scatter(msg, rx, ry, rz, cw, dsts, za, z1, na):
    e_pad, d = msg.shape
    eph = e_pad // NC
    ept = eph // NS
    nch = ept // CH
    rpt = na // NS
    mesh = plsc.VectorSubcoreMesh(core_axis_name="c", subcore_axis_name="s")

    @functools.partial(
        pl.kernel,
        mesh=mesh,
        out_type=[
            jax.ShapeDtypeStruct((NC * na, d), F32),
            jax.ShapeDtypeStruct((NC * na,), F32),
            jax.ShapeDtypeStruct((NC * na,), F32),
            jax.ShapeDtypeStruct((NC * na,), F32),
        ],
        scratch_types=[
            pltpu.VMEM_SHARED((na, d), F32),
            pltpu.VMEM_SHARED((na,), F32),
            pltpu.VMEM_SHARED((na,), F32),
            pltpu.VMEM_SHARED((na,), F32),
            pltpu.VMEM((CH, d), F32),
            pltpu.VMEM((CH,), F32),
            pltpu.VMEM((CH,), F32),
            pltpu.VMEM((CH,), F32),
            pltpu.VMEM((CH,), F32),
            pltpu.VMEM((CH,), jnp.int32),
        ],
    )
    def k(msgh, rxh, ryh, rzh, cwh, dsth, zah, z1h,
          aggo, axo, ayo, azo,
          accsh, axsh, aysh, azsh, mv, rxv, ryv, rzv, cwv, dv):
        cid = lax.axis_index("c")
        sid = lax.axis_index("s")
        r0 = sid * rpt
        rsl = pl.ds(r0, rpt)
        pltpu.sync_copy(zah.at[rsl], accsh.at[rsl])
        pltpu.sync_copy(z1h.at[rsl], axsh.at[rsl])
        pltpu.sync_copy(z1h.at[rsl], aysh.at[rsl])
        pltpu.sync_copy(z1h.at[rsl], azsh.at[rsl])
        plsc.subcore_barrier()

        @pl.loop(0, nch)
        def _(j):
            off = cid * eph + sid * ept + j * CH
            esl = pl.ds(off, CH)
            pltpu.sync_copy(dsth.at[esl], dv)
            pltpu.sync_copy(msgh.at[esl], mv)
            pltpu.sync_copy(rxh.at[esl], rxv)
            pltpu.sync_copy(ryh.at[esl], ryv)
            pltpu.sync_copy(rzh.at[esl], rzv)
            pltpu.sync_copy(cwh.at[esl], cwv)
            for i in range(CH // SL):
                sl = pl.ds(i * SL, SL)
                c = cwv[sl]
                rxv[sl] = rxv[sl] * c
                ryv[sl] = ryv[sl] * c
                rzv[sl] = rzv[sl] * c
            pltpu.sync_copy(mv, accsh.at[dv], add=True)
            pltpu.sync_copy(rxv, axsh.at[dv], add=True)
            pltpu.sync_copy(ryv, aysh.at[dv], add=True)
            pltpu.sync_copy(rzv, azsh.at[dv], add=True)

        plsc.subcore_barrier()
        o0 = cid * na + r0
        osl = pl.ds(o0, rpt)
        pltpu.sync_copy(accsh.at[rsl], aggo.at[osl])
        pltpu.sync_copy(axsh.at[rsl], axo.at[osl])
        pltpu.sync_copy(aysh.at[rsl], ayo.at[osl])
        pltpu.sync_copy(azsh.at[rsl], azo.at[osl])

    return k(msg, rx, ry, rz, cw, dsts, za, z1)


def _node_body(h, a0, a1, xx, xy, xz, cx0, cx1, cy0, cy1, cz0, cz1,
               wn1a, wn1b, bn1, wn2, bn2, lng, lnb,
               ho_o, xox_o, xoy_o, xoz_o):
    hb = h[...]
    agg = a0[...] + a1[...]
    u = _silu(_dot(hb, wn1a[...]) + _dot(agg, wn1b[...]) + bn1[...])
    ho = hb + _dot(u, wn2[...]) + bn2[...]
    mu = jnp.mean(ho, axis=1, keepdims=True)
    dev = ho - mu
    var = jnp.mean(dev * dev, axis=1, keepdims=True)
    ho_o[...] = dev * jax.lax.rsqrt(var + EPS) * lng[...] + lnb[...]
    xox_o[...] = xx[...] + cx0[...] + cx1[...]
    xoy_o[...] = xy[...] + cy0[...] + cy1[...]
    xoz_o[...] = xz[...] + cz0[...] + cz1[...]


def _node(h, a0, a1, xx, xy, xz, cx0, cx1, cy0, cy1, cz0, cz1,
          wn1a, wn1b, bn1, wn2, bn2, lng, lnb, bn, bc):
    n, d = h.shape
    nc_len = xx.shape[0]
    grid = n // bn
    assert nc_len // bc == grid

    def nb(i):
        return (i, 0)

    def n1(i):
        return (i,)

    def cb(i):
        return (0, 0)

    return pl.pallas_call(
        _node_body,
        grid=(grid,),
        in_specs=[
            pl.BlockSpec((bn, d), nb),
            pl.BlockSpec((bn, d), nb),
            pl.BlockSpec((bn, d), nb),
            pl.BlockSpec((bc,), n1),
            pl.BlockSpec((bc,), n1),
            pl.BlockSpec((bc,), n1),
            pl.BlockSpec((bc,), n1),
            pl.BlockSpec((bc,), n1),
            pl.BlockSpec((bc,), n1),
            pl.BlockSpec((bc,), n1),
            pl.BlockSpec((bc,), n1),
            pl.BlockSpec((bc,), n1),
            pl.BlockSpec((d, d), cb),
            pl.BlockSpec((d, d), cb),
            pl.BlockSpec((1, d), cb),
            pl.BlockSpec((d, d), cb),
            pl.BlockSpec((1, d), cb),
            pl.BlockSpec((1, d), cb),
            pl.BlockSpec((1, d), cb),
        ],
        out_specs=[
            pl.BlockSpec((bn, d), nb),
            pl.BlockSpec((bc,), n1),
            pl.BlockSpec((bc,), n1),
            pl.BlockSpec((bc,), n1),
        ],
        out_shape=[
            jax.ShapeDtypeStruct((n, d), F32),
            jax.ShapeDtypeStruct((nc_len,), F32),
            jax.ShapeDtypeStruct((nc_len,), F32),
            jax.ShapeDtypeStruct((nc_len,), F32),
        ],
    )(h, a0, a1, xx, xy, xz, cx0, cx1, cy0, cy1, cz0, cz1,
      wn1a, wn1b, bn1, wn2, bn2, lng, lnb)


def kernel(h, x, edge_index, edge_attr, W_e1, b_e1, W_e2, b_e2, W_n1, b_n1,
           W_n2, b_n2, W_c1, b_c1, W_c2, b_c2, ln_g, ln_b):
    n, d = h.shape
    e = edge_index.shape[1]

    e_pad = ((e + NW * CH - 1) // (NW * CH)) * (NW * CH)
    na = ((n + 1 + 1024 - 1) // 1024) * 1024

    src = edge_index[0]
    dst = edge_index[1]
    padg = jnp.zeros((e_pad - e,), jnp.int32)
    srcg = jnp.concatenate([src, padg])
    dstg = jnp.concatenate([dst, padg])
    dsts = jnp.concatenate([dst, jnp.full((e_pad - e,), n, jnp.int32)])

    w1a = W_e1[:d]
    w1b = W_e1[d:2 * d]
    wdr = W_e1[2 * d:2 * d + 1]
    w1c = W_e1[2 * d + 1:]
    be1 = b_e1.reshape(1, d)
    be2 = b_e2.reshape(1, d)
    bc1 = b_c1.reshape(1, d)
    wc2r = W_c2.reshape(1, d)
    bc2r = jnp.broadcast_to(b_c2.reshape(1, 1), (1, d))
    wn1a = W_n1[:d]
    wn1b = W_n1[d:]
    bn1 = b_n1.reshape(1, d)
    bn2 = b_n2.reshape(1, d)
    lng = ln_g.reshape(1, d)
    lnb = ln_b.reshape(1, d)

    xx = x[:, 0]
    xy = x[:, 1]
    xz = x[:, 2]
    xxp = jnp.pad(xx, (0, na - n))
    xyp = jnp.pad(xy, (0, na - n))
    xzp = jnp.pad(xz, (0, na - n))
    eap = jnp.pad(edge_attr, ((0, e_pad - e), (0, 0)))

    t1, t2 = _prep(h, w1a, w1b, be1, bn=1000)
    g1, g2, rx, ry, rz, ds1 = _sc_gather(t1, t2, xx, xy, xz, srcg, dstg,
                                         e_pad)
    msg, cw = _edge(g1, g2, eap, ds1, wdr, w1c, W_e2, be2, W_c1,
                    bc1, wc2r, bc2r, be_blk=4096)
    za = jnp.zeros((na, d), F32)
    z1 = jnp.zeros((na,), F32)
    agg, ax, ay, az = _sc_scatter(msg, rx, ry, rz, cw, dsts, za, z1, na)
    h_out, xox, xoy, xoz = _node(
        h, agg[:n], agg[na:na + n], xxp, xyp, xzp,
        ax[:na], ax[na:], ay[:na], ay[na:],
        az[:na], az[na:],
        wn1a, wn1b, bn1, W_n2, bn2, lng, lnb, bn=1000, bc=1024)
    return (h_out, jnp.stack([xox[:n], xoy[:n], xoz[:n]], axis=1))


# speedup vs baseline: 6.6016x; 2.9749x over previous
import dataclasses
import functools

import jax
import jax.numpy as jnp
from jax import lax
from jax.experimental import pallas as pl
from jax.experimental.pallas import tpu as pltpu
from jax.experimental.pallas import tpu_sc as plsc

F32 = jnp.float32
EPS = 1e-5

NC = 2
NS = 16
NW = NC * NS
CH = 128
SL = 16

_HIGH = jax.lax.Precision.HIGHEST


def _sc_params():
    cp = pltpu.CompilerParams()
    if "needs_layout_passes" in pltpu.CompilerParams.__dataclass_fields__:
        cp = dataclasses.replace(cp, needs_layout_passes=False)
    return cp


def _dot(a, b):
    return jnp.dot(a, b, preferred_element_type=F32)


def _silu(v):
    return v * jax.nn.sigmoid(v)


def _prep_body(h, w1a, w1b, be1, t1_o, t2_o):
    hb = h[...]
    t1_o[...] = _dot(hb, w1a[...])
    t2_o[...] = _dot(hb, w1b[...]) + be1[...]


def _prep(h, w1a, w1b, be1, bn):
    n, d = h.shape
    grid = n // bn
    return pl.pallas_call(
        _prep_body,
        grid=(grid,),
        in_specs=[
            pl.BlockSpec((bn, d), lambda i: (i, 0)),
            pl.BlockSpec((d, d), lambda i: (0, 0)),
            pl.BlockSpec((d, d), lambda i: (0, 0)),
            pl.BlockSpec((1, d), lambda i: (0, 0)),
        ],
        out_specs=[
            pl.BlockSpec((bn, d), lambda i: (i, 0)),
            pl.BlockSpec((bn, d), lambda i: (i, 0)),
        ],
        out_shape=[
            jax.ShapeDtypeStruct((n, d), F32),
            jax.ShapeDtypeStruct((n, d), F32),
        ],
    )(h, w1a, w1b, be1)


def _sc_gather(t1, t2, xx, xy, xz, srcg, dstg, e_pad):
    CHG = 64
    npad, d = t1.shape
    n = xx.shape[0]
    ept = e_pad // NS
    nch = ept // CHG
    rpt = npad // NS
    mesh = plsc.VectorSubcoreMesh(core_axis_name="c", subcore_axis_name="s")

    @functools.partial(
        pl.kernel,
        mesh=mesh,
        out_type=[
            jax.ShapeDtypeStruct((e_pad, d), F32),
            jax.ShapeDtypeStruct((e_pad, d), F32),
            jax.ShapeDtypeStruct((e_pad,), F32),
            jax.ShapeDtypeStruct((e_pad,), F32),
            jax.ShapeDtypeStruct((e_pad,), F32),
            jax.ShapeDtypeStruct((e_pad,), F32),
        ],
        scratch_types=[
            pltpu.VMEM_SHARED((npad, d), F32),
            pltpu.VMEM((n,), F32),
            pltpu.VMEM((n,), F32),
            pltpu.VMEM((n,), F32),
            pltpu.VMEM((2, CHG), jnp.int32),
            pltpu.VMEM((2, CHG), jnp.int32),
            pltpu.VMEM((2, CHG, d), F32),
            pltpu.VMEM((2, CHG), F32),
            pltpu.VMEM((2, CHG), F32),
            pltpu.VMEM((2, CHG), F32),
            pltpu.VMEM((2, CHG), F32),
            pltpu.SemaphoreType.DMA,
            pltpu.SemaphoreType.DMA,
            pltpu.SemaphoreType.DMA,
            pltpu.SemaphoreType.DMA,
            pltpu.SemaphoreType.DMA,
            pltpu.SemaphoreType.DMA,
        ],
        compiler_params=_sc_params(),
    )
    def k(t1h, t2h, xxh, xyh, xzh, sh, dh,
          g1o, g2o, rxo, ryo, rzo, dso,
          tabsh, xxv, xyv, xzv, mv, av, gv, rxv, ryv, rzv, dsv,
          si0, si1, sg0, sg1, sw0, sw1):
        cid = lax.axis_index("c")
        sid = lax.axis_index("s")
        si = (si0, si1)
        sg = (sg0, sg1)
        sw = (sw0, sw1)
        tbase = sid * ept
        rsl = pl.ds(sid * rpt, rpt)

        @pl.when(cid == 0)
        def _():
            pltpu.sync_copy(t1h.at[rsl], tabsh.at[rsl])

        @pl.when(cid == 1)
        def _():
            pltpu.sync_copy(t2h.at[rsl], tabsh.at[rsl])

        pltpu.sync_copy(xxh, xxv)
        pltpu.sync_copy(xyh, xyv)
        pltpu.sync_copy(xzh, xzv)
        plsc.subcore_barrier()

        def run(main_h, aux_h, g_out, rel_parity, main_is_src):
            def rel_wb(b, off):
                esl = pl.ds(off, CHG)
                return [
                    (rxv.at[b], rxo.at[esl]),
                    (ryv.at[b], ryo.at[esl]),
                    (rzv.at[b], rzo.at[esl]),
                    (dsv.at[b], dso.at[esl]),
                ]

            for b in range(2):
                esl = pl.ds(tbase + b * CHG, CHG)
                pltpu.async_copy(main_h.at[esl], mv.at[b], si[b])
                if b == rel_parity:
                    pltpu.async_copy(aux_h.at[esl], av.at[b], si[b])

            @pl.loop(0, nch, step=2)
            def _(jj):
                for b in range(2):
                    do_rel = b == rel_parity
                    j = jj + b
                    off = tbase + j * CHG
                    esl = pl.ds(off, CHG)

                    @pl.when(jj >= 2)
                    def _():
                        pltpu.make_async_copy(
                            gv.at[b], g_out.at[esl], sw[b]).wait()
                        if do_rel:
                            for s, t in rel_wb(b, off):
                                pltpu.make_async_copy(s, t, sw[b]).wait()

                    pltpu.make_async_copy(
                        main_h.at[esl], mv.at[b], si[b]).wait()
                    if do_rel:
                        pltpu.make_async_copy(
                            aux_h.at[esl], av.at[b], si[b]).wait()

                    cg = pltpu.async_copy(
                        tabsh.at[mv.at[b]], gv.at[b], sg[b])

                    if do_rel:
                        if main_is_src:
                            svb, dvb = mv.at[b], av.at[b]
                        else:
                            svb, dvb = av.at[b], mv.at[b]
                        for i in range(CHG // SL):
                            sl = pl.ds(i * SL, SL)
                            siv = svb[sl]
                            div = dvb[sl]
                            rx = (plsc.load_gather(xxv, [siv])
                                  - plsc.load_gather(xxv, [div]))
                            ry = (plsc.load_gather(xyv, [siv])
                                  - plsc.load_gather(xyv, [div]))
                            rz = (plsc.load_gather(xzv, [siv])
                                  - plsc.load_gather(xzv, [div]))
                            rxv.at[b][sl] = rx
                            ryv.at[b][sl] = ry
                            rzv.at[b][sl] = rz
                            dsv.at[b][sl] = rx * rx + ry * ry + rz * rz

                    cg.wait()
                    pltpu.async_copy(gv.at[b], g_out.at[esl], sw[b])
                    if do_rel:
                        for s, t in rel_wb(b, off):
                            pltpu.async_copy(s, t, sw[b])

                    @pl.when(j + 2 < nch)
                    def _():
                        esl2 = pl.ds(off + 2 * CHG, CHG)
                        pltpu.async_copy(main_h.at[esl2], mv.at[b], si[b])
                        if do_rel:
                            pltpu.async_copy(aux_h.at[esl2], av.at[b], si[b])

            for b in range(2):
                off = tbase + (nch - 2 + b) * CHG
                pltpu.make_async_copy(
                    gv.at[b], g_out.at[pl.ds(off, CHG)], sw[b]).wait()
                if b == rel_parity:
                    for s, t in rel_wb(b, off):
                        pltpu.make_async_copy(s, t, sw[b]).wait()

        @pl.when(cid == 0)
        def _():
            run(sh, dh, g1o, 0, True)

        @pl.when(cid == 1)
        def _():
            run(dh, sh, g2o, 1, False)

    return k(t1, t2, xx, xy, xz, srcg, dstg)


def _edge_body(g1, g2, ea, ds1, wdr, w1c, we2, be2, wc1, bc1, wc2r, bc2r,
               msg_o, cw_o):
    dsb = ds1[...]
    dsc = jnp.reshape(dsb, (dsb.shape[0], 1))
    pre = g1[...] + g2[...] + dsc * wdr[...] + _dot(ea[...], w1c[...])
    m = _silu(pre)
    msg = _silu(_dot(m, we2[...]) + be2[...])
    t = _silu(_dot(msg, wc1[...]) + bc1[...])
    cw = jnp.sum(t * wc2r[...], axis=1) + bc2r[0, 0]
    msg_o[...] = msg
    cw_o[...] = cw


def _edge(g1, g2, ea, ds1, wdr, w1c, we2, be2, wc1, bc1, wc2r, bc2r,
          be_blk):
    e_pad, d = g1.shape
    ed = ea.shape[1]
    grid = e_pad // be_blk

    def eb(i):
        return (i, 0)

    def e1(i):
        return (i,)

    def cb(i):
        return (0, 0)

    return pl.pallas_call(
        _edge_body,
        grid=(grid,),
        in_specs=[
            pl.BlockSpec((be_blk, d), eb),
            pl.BlockSpec((be_blk, d), eb),
            pl.BlockSpec((be_blk, ed), eb),
            pl.BlockSpec((be_blk,), e1),
            pl.BlockSpec((1, d), cb),
            pl.BlockSpec((ed, d), cb),
            pl.BlockSpec((d, d), cb),
            pl.BlockSpec((1, d), cb),
            pl.BlockSpec((d, d), cb),
            pl.BlockSpec((1, d), cb),
            pl.BlockSpec((1, d), cb),
            pl.BlockSpec((1, d), cb),
        ],
        out_specs=[
            pl.BlockSpec((be_blk, d), eb),
            pl.BlockSpec((be_blk,), e1),
        ],
        out_shape=[
            jax.ShapeDtypeStruct((e_pad, d), F32),
            jax.ShapeDtypeStruct((e_pad,), F32),
        ],
    )(g1, g2, ea, ds1, wdr, w1c, we2, be2, wc1, bc1, wc2r, bc2r)


def _sc_scatter(msg, rx, ry, rz, cw, dsts, za, z1, na):
    e_pad, d = msg.shape
    eph = e_pad // NC
    ept = eph // NS
    nch = ept // CH
    rpt = na // NS
    mesh = plsc.VectorSubcoreMesh(core_axis_name="c", subcore_axis_name="s")

    @functools.partial(
        pl.kernel,
        mesh=mesh,
        out_type=[
            jax.ShapeDtypeStruct((NC * na, d), F32),
            jax.ShapeDtypeStruct((NC * na,), F32),
            jax.ShapeDtypeStruct((NC * na,), F32),
            jax.ShapeDtypeStruct((NC * na,), F32),
        ],
        scratch_types=[
            pltpu.VMEM_SHARED((na, d), F32),
            pltpu.VMEM_SHARED((na,), F32),
            pltpu.VMEM_SHARED((na,), F32),
            pltpu.VMEM_SHARED((na,), F32),
            pltpu.VMEM((2, CH, d), F32),
            pltpu.VMEM((2, CH), F32),
            pltpu.VMEM((2, CH), F32),
            pltpu.VMEM((2, CH), F32),
            pltpu.VMEM((2, CH), F32),
            pltpu.VMEM((2, CH), jnp.int32),
            pltpu.SemaphoreType.DMA,
            pltpu.SemaphoreType.DMA,
            pltpu.SemaphoreType.DMA,
            pltpu.SemaphoreType.DMA,
        ],
        compiler_params=_sc_params(),
    )
    def k(msgh, rxh, ryh, rzh, cwh, dsth, zah, z1h,
          aggo, axo, ayo, azo,
          accsh, axsh, aysh, azsh, mv, rxv, ryv, rzv, cwv, dv,
          sr0, sr1, ss0, ss1):
        cid = lax.axis_index("c")
        sid = lax.axis_index("s")
        sr = (sr0, sr1)
        ss = (ss0, ss1)
        r0 = sid * rpt
        rsl = pl.ds(r0, rpt)
        pltpu.sync_copy(zah.at[rsl], accsh.at[rsl])
        pltpu.sync_copy(z1h.at[rsl], axsh.at[rsl])
        pltpu.sync_copy(z1h.at[rsl], aysh.at[rsl])
        pltpu.sync_copy(z1h.at[rsl], azsh.at[rsl])
        plsc.subcore_barrier()

        tbase = cid * eph + sid * ept

        def rd_list(b, off):
            esl = pl.ds(off, CH)
            return [
                (dsth.at[esl], dv.at[b]),
                (msgh.at[esl], mv.at[b]),
                (rxh.at[esl], rxv.at[b]),
                (ryh.at[esl], ryv.at[b]),
                (rzh.at[esl], rzv.at[b]),
                (cwh.at[esl], cwv.at[b]),
            ]

        def sa_list(b):
            return [
                (mv.at[b], accsh.at[dv.at[b]]),
                (rxv.at[b], axsh.at[dv.at[b]]),
                (ryv.at[b], aysh.at[dv.at[b]]),
                (rzv.at[b], azsh.at[dv.at[b]]),
            ]

        for b in range(2):
            for s, t in rd_list(b, tbase + b * CH):
                pltpu.async_copy(s, t, sr[b])

        @pl.loop(0, nch, step=2)
        def _(jj):
            for b in range(2):
                j = jj + b
                off = tbase + j * CH
                for s, t in rd_list(b, off):
                    pltpu.make_async_copy(s, t, sr[b]).wait()
                rb = rxv.at[b]
                yb = ryv.at[b]
                zb = rzv.at[b]
                cb = cwv.at[b]
                for i in range(CH // SL):
                    sl = pl.ds(i * SL, SL)
                    c = cb[sl]
                    rb[sl] = rb[sl] * c
                    yb[sl] = yb[sl] * c
                    zb[sl] = zb[sl] * c
                for s, t in sa_list(b):
                    pltpu.async_copy(s, t, ss[b], add=True)

                @pl.when(j + 2 < nch)
                def _():
                    for s, t in sa_list(b):
                        pltpu.make_async_copy(s, t, ss[b]).wait()
                    for s, t in rd_list(b, off + 2 * CH):
                        pltpu.async_copy(s, t, sr[b])

        for b in range(2):
            for s, t in sa_list(b):
                pltpu.make_async_copy(s, t, ss[b]).wait()

        plsc.subcore_barrier()
        o0 = cid * na + r0
        osl = pl.ds(o0, rpt)
        pltpu.sync_copy(accsh.at[rsl], aggo.at[osl])
        pltpu.sync_copy(axsh.at[rsl], axo.at[osl])
        pltpu.sync_copy(aysh.at[rsl], ayo.at[osl])
        pltpu.sync_copy(azsh.at[rsl], azo.at[osl])

    return k(msg, rx, ry, rz, cw, dsts, za, z1)


def _node_body(h, a0, a1, xx, xy, xz, cx0, cx1, cy0, cy1, cz0, cz1,
               wn1a, wn1b, bn1, wn2, bn2, lng, lnb,
               ho_o, xox_o, xoy_o, xoz_o):
    hb = h[...]
    agg = a0[...] + a1[...]
    u = _silu(_dot(hb, wn1a[...]) + _dot(agg, wn1b[...]) + bn1[...])
    ho = hb + _dot(u, wn2[...]) + bn2[...]
    mu = jnp.mean(ho, axis=1, keepdims=True)
    dev = ho - mu
    var = jnp.mean(dev * dev, axis=1, keepdims=True)
    ho_o[...] = dev * jax.lax.rsqrt(var + EPS) * lng[...] + lnb[...]
    xox_o[...] = xx[...] + cx0[...] + cx1[...]
    xoy_o[...] = xy[...] + cy0[...] + cy1[...]
    xoz_o[...] = xz[...] + cz0[...] + cz1[...]


def _node(h, a0, a1, xx, xy, xz, cx0, cx1, cy0, cy1, cz0, cz1,
          wn1a, wn1b, bn1, wn2, bn2, lng, lnb, bn, bc):
    n, d = h.shape
    nc_len = xx.shape[0]
    grid = n // bn
    assert nc_len // bc == grid

    def nb(i):
        return (i, 0)

    def n1(i):
        return (i,)

    def cb(i):
        return (0, 0)

    return pl.pallas_call(
        _node_body,
        grid=(grid,),
        in_specs=[
            pl.BlockSpec((bn, d), nb),
            pl.BlockSpec((bn, d), nb),
            pl.BlockSpec((bn, d), nb),
            pl.BlockSpec((bc,), n1),
            pl.BlockSpec((bc,), n1),
            pl.BlockSpec((bc,), n1),
            pl.BlockSpec((bc,), n1),
            pl.BlockSpec((bc,), n1),
            pl.BlockSpec((bc,), n1),
            pl.BlockSpec((bc,), n1),
            pl.BlockSpec((bc,), n1),
            pl.BlockSpec((bc,), n1),
            pl.BlockSpec((d, d), cb),
            pl.BlockSpec((d, d), cb),
            pl.BlockSpec((1, d), cb),
            pl.BlockSpec((d, d), cb),
            pl.BlockSpec((1, d), cb),
            pl.BlockSpec((1, d), cb),
            pl.BlockSpec((1, d), cb),
        ],
        out_specs=[
            pl.BlockSpec((bn, d), nb),
            pl.BlockSpec((bc,), n1),
            pl.BlockSpec((bc,), n1),
            pl.BlockSpec((bc,), n1),
        ],
        out_shape=[
            jax.ShapeDtypeStruct((n, d), F32),
            jax.ShapeDtypeStruct((nc_len,), F32),
            jax.ShapeDtypeStruct((nc_len,), F32),
            jax.ShapeDtypeStruct((nc_len,), F32),
        ],
    )(h, a0, a1, xx, xy, xz, cx0, cx1, cy0, cy1, cz0, cz1,
      wn1a, wn1b, bn1, wn2, bn2, lng, lnb)


def kernel(h, x, edge_index, edge_attr, W_e1, b_e1, W_e2, b_e2, W_n1, b_n1,
           W_n2, b_n2, W_c1, b_c1, W_c2, b_c2, ln_g, ln_b):
    n, d = h.shape
    e = edge_index.shape[1]

    quant = 2 * NW * CH
    e_pad = ((e + quant - 1) // quant) * quant
    na = ((n + 1 + 1024 - 1) // 1024) * 1024

    src = edge_index[0]
    dst = edge_index[1]
    padg = jnp.zeros((e_pad - e,), jnp.int32)
    srcg = jnp.concatenate([src, padg])
    dstg = jnp.concatenate([dst, padg])
    dsts = jnp.concatenate([dst, jnp.full((e_pad - e,), n, jnp.int32)])

    w1a = W_e1[:d]
    w1b = W_e1[d:2 * d]
    wdr = W_e1[2 * d:2 * d + 1]
    w1c = W_e1[2 * d + 1:]
    be1 = b_e1.reshape(1, d)
    be2 = b_e2.reshape(1, d)
    bc1 = b_c1.reshape(1, d)
    wc2r = W_c2.reshape(1, d)
    bc2r = jnp.broadcast_to(b_c2.reshape(1, 1), (1, d))
    wn1a = W_n1[:d]
    wn1b = W_n1[d:]
    bn1 = b_n1.reshape(1, d)
    bn2 = b_n2.reshape(1, d)
    lng = ln_g.reshape(1, d)
    lnb = ln_b.reshape(1, d)

    xx = x[:, 0]
    xy = x[:, 1]
    xz = x[:, 2]
    xxp = jnp.pad(xx, (0, na - n))
    xyp = jnp.pad(xy, (0, na - n))
    xzp = jnp.pad(xz, (0, na - n))
    eap = jnp.pad(edge_attr, ((0, e_pad - e), (0, 0)))

    t1, t2 = _prep(h, w1a, w1b, be1, bn=1000)
    npad = ((n + 127) // 128) * 128
    t1p = jnp.pad(t1, ((0, npad - n), (0, 0)))
    t2p = jnp.pad(t2, ((0, npad - n), (0, 0)))
    g1, g2, rx, ry, rz, ds1 = _sc_gather(t1p, t2p, xx, xy, xz, srcg, dstg,
                                         e_pad)
    msg, cw = _edge(g1, g2, eap, ds1, wdr, w1c, W_e2, be2, W_c1,
                    bc1, wc2r, bc2r, be_blk=4096)
    za = jnp.zeros((na, d), F32)
    z1 = jnp.zeros((na,), F32)
    agg, ax, ay, az = _sc_scatter(msg, rx, ry, rz, cw, dsts, za, z1, na)
    h_out, xox, xoy, xoz = _node(
        h, agg[:n], agg[na:na + n], xxp, xyp, xzp,
        ax[:na], ax[na:], ay[:na], ay[na:],
        az[:na], az[na:],
        wn1a, wn1b, bn1, W_n2, bn2, lng, lnb, bn=1000, bc=1024)
    return (h_out, jnp.stack([xox[:n], xoy[:n], xoz[:n]], axis=1))
